# baseline (speedup 1.0000x reference)
import functools

import jax
import jax.numpy as jnp
from jax import lax
from jax.experimental import pallas as pl
from jax.experimental.pallas import tpu as pltpu
from jax.experimental.pallas import tpu_sc as plsc

_NC, _NS = 2, 16
_NW = _NC * _NS


def _sc_broadcast(B, S, D, dtype, W_pos):
    rows_w = S // _NW
    CH = min(rows_w, 64)
    n_ch = rows_w // CH
    mesh = plsc.VectorSubcoreMesh(
        core_axis_name="c", subcore_axis_name="s",
        num_cores=_NC, num_subcores=_NS)

    @functools.partial(
        pl.kernel,
        out_type=jax.ShapeDtypeStruct((B, S, D), dtype),
        mesh=mesh,
        scratch_types=[
            pltpu.VMEM((CH, D), dtype),
            pltpu.SemaphoreType.DMA,
        ],
    )
    def k(w_hbm, out_hbm, buf, sem):
        wid = lax.axis_index("s") * _NC + lax.axis_index("c")
        base = wid * rows_w
        for c in range(n_ch):
            r0 = base + c * CH
            pltpu.sync_copy(w_hbm.at[pl.ds(r0, CH)], buf)
            for b in range(B):
                pltpu.async_copy(buf, out_hbm.at[b, pl.ds(r0, CH)], sem)
            for b in range(B):
                pltpu.make_async_copy(buf, out_hbm.at[0, pl.ds(r0, CH)], sem).wait()

    return k(W_pos)


def kernel(tokens, token_embed, W_pos):
    B, S, D = token_embed.shape
    pos = _sc_broadcast(B, S, D, W_pos.dtype, W_pos)
    return (pos, token_embed)


# speedup vs baseline: 1.0014x; 1.0014x over previous
import functools

import jax
import jax.numpy as jnp
from jax import lax
from jax.experimental import pallas as pl
from jax.experimental.pallas import tpu as pltpu
from jax.experimental.pallas import tpu_sc as plsc

_NC, _NS = 2, 16
_NW = _NC * _NS


def _sc_broadcast(B, S, D, dtype, W_pos):
    rows_w = S // _NW
    CH = min(rows_w, 64)
    n_ch = rows_w // CH
    mesh = plsc.VectorSubcoreMesh(
        core_axis_name="c", subcore_axis_name="s",
        num_cores=_NC, num_subcores=_NS)

    @functools.partial(
        pl.kernel,
        out_type=jax.ShapeDtypeStruct((B, S, D), dtype),
        mesh=mesh,
        scratch_types=[
            pltpu.VMEM((CH, D), dtype),
            pltpu.SemaphoreType.DMA,
        ],
    )
    def k(w_hbm, out_hbm, buf, sem):
        wid = lax.axis_index("s") * _NC + lax.axis_index("c")
        base = wid * rows_w
        for c in range(n_ch):
            r0 = base + c * CH
            pltpu.sync_copy(w_hbm.at[pl.ds(r0, CH)], buf)
            for b in range(B):
                pltpu.async_copy(buf, out_hbm.at[b, pl.ds(r0, CH)], sem)
            for b in range(B):
                pltpu.make_async_copy(buf, out_hbm.at[0, pl.ds(r0, CH)], sem).wait()

    return k(W_pos)


def _copy_body(x_ref, o_ref):
    o_ref[...] = x_ref[...]


def _tc_copy(x):
    B, S, D = x.shape
    CBS = 512
    flat = x.reshape(B * S, D)
    out = pl.pallas_call(
        _copy_body,
        grid=(B * S // CBS,),
        in_specs=[pl.BlockSpec((CBS, D), lambda i: (i, 0))],
        out_specs=pl.BlockSpec((CBS, D), lambda i: (i, 0)),
        out_shape=jax.ShapeDtypeStruct((B * S, D), x.dtype),
    )(flat)
    return out.reshape(B, S, D)


def kernel(tokens, token_embed, W_pos):
    B, S, D = token_embed.shape
    pos = _sc_broadcast(B, S, D, W_pos.dtype, W_pos)
    tok = _tc_copy(token_embed)
    return (pos, tok)


# speedup vs baseline: 1.2169x; 1.2152x over previous
import jax
import jax.numpy as jnp
from jax.experimental import pallas as pl


def _body(w_ref, t_ref, pos_ref, tok_ref):
    pos_ref[...] = jnp.broadcast_to(w_ref[...][None, :, :], pos_ref.shape)
    tok_ref[...] = t_ref[...]


def kernel(tokens, token_embed, W_pos):
    B, S, D = token_embed.shape
    CH = 128
    TCH = CH * B
    tok_flat = token_embed.reshape(B * S, D)
    pos, tok = pl.pallas_call(
        _body,
        grid=(S // CH,),
        in_specs=[
            pl.BlockSpec((CH, D), lambda i: (i, 0)),
            pl.BlockSpec((TCH, D), lambda i: (i, 0)),
        ],
        out_specs=[
            pl.BlockSpec((B, CH, D), lambda i: (0, i, 0)),
            pl.BlockSpec((TCH, D), lambda i: (i, 0)),
        ],
        out_shape=[
            jax.ShapeDtypeStruct((B, S, D), W_pos.dtype),
            jax.ShapeDtypeStruct((B * S, D), token_embed.dtype),
        ],
    )(W_pos, tok_flat)
    return (pos, tok.reshape(B, S, D))


# speedup vs baseline: 1.2771x; 1.0494x over previous
import jax
import jax.numpy as jnp
from jax.experimental import pallas as pl


def _body(w_ref, t_ref, pos_ref, tok_ref):
    pos_ref[...] = jnp.broadcast_to(w_ref[...][None, :, :], pos_ref.shape)
    tok_ref[...] = t_ref[...]


def kernel(tokens, token_embed, W_pos):
    B, S, D = token_embed.shape
    CH = 256
    TCH = CH * B
    tok_flat = token_embed.reshape(B * S, D)
    pos, tok = pl.pallas_call(
        _body,
        grid=(S // CH,),
        in_specs=[
            pl.BlockSpec((CH, D), lambda i: (i, 0)),
            pl.BlockSpec((TCH, D), lambda i: (i, 0)),
        ],
        out_specs=[
            pl.BlockSpec((B, CH, D), lambda i: (0, i, 0)),
            pl.BlockSpec((TCH, D), lambda i: (i, 0)),
        ],
        out_shape=[
            jax.ShapeDtypeStruct((B, S, D), W_pos.dtype),
            jax.ShapeDtypeStruct((B * S, D), token_embed.dtype),
        ],
    )(W_pos, tok_flat)
    return (pos, tok.reshape(B, S, D))


# speedup vs baseline: 1.3228x; 1.0358x over previous
import jax
import jax.numpy as jnp
from jax.experimental import pallas as pl


def _body(w_ref, t_ref, pos_ref, tok_ref):
    pos_ref[...] = jnp.broadcast_to(w_ref[...][None, :, :], pos_ref.shape)
    tok_ref[...] = t_ref[...]


def kernel(tokens, token_embed, W_pos):
    B, S, D = token_embed.shape
    CH = 512
    TCH = CH * B
    tok_flat = token_embed.reshape(B * S, D)
    pos, tok = pl.pallas_call(
        _body,
        grid=(S // CH,),
        in_specs=[
            pl.BlockSpec((CH, D), lambda i: (i, 0)),
            pl.BlockSpec((TCH, D), lambda i: (i, 0)),
        ],
        out_specs=[
            pl.BlockSpec((B, CH, D), lambda i: (0, i, 0)),
            pl.BlockSpec((TCH, D), lambda i: (i, 0)),
        ],
        out_shape=[
            jax.ShapeDtypeStruct((B, S, D), W_pos.dtype),
            jax.ShapeDtypeStruct((B * S, D), token_embed.dtype),
        ],
    )(W_pos, tok_flat)
    return (pos, tok.reshape(B, S, D))
